# baseline (speedup 1.0000x reference)
import jax
import jax.numpy as jnp
from jax.experimental import pallas as pl
from jax.experimental.pallas import tpu as pltpu

T, B, N = 4, 8, 512
BN = B * N
IN_DIM, HID, OUT = 128, 128, 64

BI = 512
NI = BN // BI


def _quantize(i, m_ref, a_ref, a8t, deg):
    a = a_ref[0]
    a8t[:, pl.ds(i * BI, BI)] = a.astype(jnp.bfloat16).T.astype(jnp.int8)
    mi = m_ref[0, 0, pl.ds(i * BI, BI)]
    part = jnp.dot(mi[None, :], a, preferred_element_type=jnp.float32)

    @pl.when(i == 0)
    def _():
        deg[...] = part

    @pl.when(i > 0)
    def _():
        deg[...] += part


def _conv1_block(i, a8t, g1b, g2b, dis, b1_ref, w2_ref):
    ab = a8t[pl.ds(i * BI, BI), :].astype(jnp.bfloat16)
    part = jax.lax.dot_general(ab, g1b[...],
                               (((1,), (0,)), ((), ())),
                               preferred_element_type=jnp.float32)
    db = dis[0, pl.ds(i * BI, BI)]
    gj = g1b[pl.ds(i * BI, BI), :].astype(jnp.float32)
    h1c = jnp.maximum((part + gj) * db[:, None] + b1_ref[...], 0.0)
    h2 = jax.lax.dot_general(h1c, w2_ref[...], (((1,), (1,)), ((), ())),
                             preferred_element_type=jnp.float32)
    g2b[pl.ds(i * BI, BI), :] = (h2 * db[:, None]).astype(jnp.bfloat16)


def _conv2_block(i, a8t, g2b, dis, b2_ref, wfc_ref, bfc_ref, out_ref):
    ab = a8t[pl.ds(i * BI, BI), :].astype(jnp.bfloat16)
    part = jax.lax.dot_general(ab, g2b[...],
                               (((1,), (0,)), ((), ())),
                               preferred_element_type=jnp.float32)
    db = dis[0, pl.ds(i * BI, BI)]
    gj = g2b[pl.ds(i * BI, BI), :].astype(jnp.float32)
    h2c = jnp.maximum((part + gj) * db[:, None] + b2_ref[...], 0.0)
    of = jax.lax.dot_general(h2c, wfc_ref[...], (((1,), (1,)), ((), ())),
                             preferred_element_type=jnp.float32)
    out_ref[0] = jnp.where(db[:, None] > 0, of + bfc_ref[...], 0.0)


def _fused_kernel(m_ref, mp_ref, a_ref, x_ref, w1_ref, b1_ref, w2_ref,
                  b2_ref, wfc_ref, bfc_ref, out_ref,
                  a8tA, a8tB, g1b, g2b, deg, dis):
    u = pl.program_id(0)
    p = pl.program_id(1)

    @pl.when(jnp.logical_and(u > 0, p == 0))
    def _():
        m = mp_ref[0, 0]
        d = deg[0] * m + m
        dis[...] = jnp.where(d > 0, jax.lax.rsqrt(d), 0.0)[None]
        h = jax.lax.dot_general(x_ref[0], w1_ref[...],
                                (((1,), (1,)), ((), ())),
                                preferred_element_type=jnp.float32)
        g1b[...] = (h * dis[0][:, None]).astype(jnp.bfloat16)

    @pl.when(jnp.logical_and(u < T, p % 2 == 0))
    def _():
        i = p // 2

        @pl.when(u % 2 == 0)
        def _():
            _quantize(i, m_ref, a_ref, a8tA, deg)

        @pl.when(u % 2 == 1)
        def _():
            _quantize(i, m_ref, a_ref, a8tB, deg)

    @pl.when(jnp.logical_and(u > 0, p < NI))
    def _():
        @pl.when(u % 2 == 1)
        def _():
            _conv1_block(p, a8tA, g1b, g2b, dis, b1_ref, w2_ref)

        @pl.when(u % 2 == 0)
        def _():
            _conv1_block(p, a8tB, g1b, g2b, dis, b1_ref, w2_ref)

    @pl.when(jnp.logical_and(u > 0, p >= NI))
    def _():
        i = p - NI

        @pl.when(u % 2 == 1)
        def _():
            _conv2_block(i, a8tA, g2b, dis, b2_ref, wfc_ref, bfc_ref, out_ref)

        @pl.when(u % 2 == 0)
        def _():
            _conv2_block(i, a8tB, g2b, dis, b2_ref, wfc_ref, bfc_ref, out_ref)


def kernel(big_batch_positions, big_batched_adjacency_pruned, ego_mask_batch,
           W1, b1, W2, b2, Wfc, bfc):
    x = big_batch_positions
    A = big_batched_adjacency_pruned
    m = jnp.transpose(ego_mask_batch, (1, 0, 2)).reshape(T, 1, BN)
    m = m.astype(jnp.float32)
    b1r = b1.reshape(1, HID)
    b2r = b2.reshape(1, HID)
    bfcr = bfc.reshape(1, OUT)

    tq = lambda u: jnp.minimum(u, T - 1)
    tc = lambda u: jnp.maximum(u - 1, 0)

    out = pl.pallas_call(
        _fused_kernel, grid=(T + 1, 2 * NI),
        in_specs=[
            pl.BlockSpec((1, 1, BN), lambda u, p: (tq(u), 0, 0)),
            pl.BlockSpec((1, 1, BN), lambda u, p: (tc(u), 0, 0)),
            pl.BlockSpec((1, BI, BN),
                         lambda u, p: (tq(u), jnp.minimum(p // 2, NI - 1), 0)),
            pl.BlockSpec((1, BN, IN_DIM), lambda u, p: (tc(u), 0, 0)),
            pl.BlockSpec((HID, IN_DIM), lambda u, p: (0, 0)),
            pl.BlockSpec((1, HID), lambda u, p: (0, 0)),
            pl.BlockSpec((HID, HID), lambda u, p: (0, 0)),
            pl.BlockSpec((1, HID), lambda u, p: (0, 0)),
            pl.BlockSpec((OUT, HID), lambda u, p: (0, 0)),
            pl.BlockSpec((1, OUT), lambda u, p: (0, 0)),
        ],
        out_specs=pl.BlockSpec(
            (1, BI, OUT),
            lambda u, p: (tc(u), jnp.clip(p - NI, 0, NI - 1), 0)),
        out_shape=jax.ShapeDtypeStruct((T, BN, OUT), jnp.float32),
        scratch_shapes=[
            pltpu.VMEM((BN, BN), jnp.int8),
            pltpu.VMEM((BN, BN), jnp.int8),
            pltpu.VMEM((BN, HID), jnp.bfloat16),
            pltpu.VMEM((BN, HID), jnp.bfloat16),
            pltpu.VMEM((1, BN), jnp.float32),
            pltpu.VMEM((1, BN), jnp.float32),
        ],
        compiler_params=pltpu.CompilerParams(
            dimension_semantics=("arbitrary", "arbitrary"),
            vmem_limit_bytes=63 * 1024 * 1024),
    )(m, m, A, x, W1, b1r, W2, b2r, Wfc, bfcr)

    h_stack = out.reshape(T, B, N, OUT)
    return jnp.transpose(h_stack, (1, 2, 0, 3))


# speedup vs baseline: 1.3186x; 1.3186x over previous
import jax
import jax.numpy as jnp
from jax.experimental import pallas as pl
from jax.experimental.pallas import tpu as pltpu

T, B, N = 4, 8, 512
BN = B * N
IN_DIM, HID, OUT = 128, 128, 64

BI = 512
NI = BN // BI
BJ = 1024
NJ = BN // BJ


def _fused_kernel(m_ref, a_ref, x_ref, w1_ref, b1_ref, w2_ref, b2_ref,
                  wfc_ref, bfc_ref, out_ref, a8t, g1b, g2b, deg, dis):
    j = pl.program_id(1)

    @pl.when(j < NI)
    def _():
        i = j
        a = a_ref[0]
        a8t[:, pl.ds(i * BI, BI)] = a.astype(jnp.bfloat16).T
        mi = m_ref[0, 0, pl.ds(i * BI, BI)]
        part = jnp.dot(mi[None, :], a, preferred_element_type=jnp.float32)

        @pl.when(i == 0)
        def _():
            deg[...] = part

        @pl.when(i > 0)
        def _():
            deg[...] += part

        @pl.when(i == NI - 1)
        def _():
            m = m_ref[0, 0]
            d = deg[0] * m + m
            dis[...] = jnp.where(d > 0, jax.lax.rsqrt(d), 0.0)[None]
            h = jax.lax.dot_general(x_ref[0], w1_ref[...],
                                    (((1,), (1,)), ((), ())),
                                    preferred_element_type=jnp.float32)
            g1b[...] = (h * dis[0][:, None]).astype(jnp.bfloat16)

    @pl.when(jnp.logical_and(j >= NI, j < NI + NJ))
    def _():
        i = j - NI
        part = jax.lax.dot_general(a8t[pl.ds(i * BJ, BJ), :], g1b[...],
                                   (((1,), (0,)), ((), ())),
                                   preferred_element_type=jnp.float32)
        db = dis[0, pl.ds(i * BJ, BJ)]
        gj = g1b[pl.ds(i * BJ, BJ), :].astype(jnp.float32)
        h1c = jnp.maximum((part + gj) * db[:, None] + b1_ref[...], 0.0)
        h2 = jax.lax.dot_general(h1c, w2_ref[...], (((1,), (1,)), ((), ())),
                                 preferred_element_type=jnp.float32)
        g2b[pl.ds(i * BJ, BJ), :] = (h2 * db[:, None]).astype(jnp.bfloat16)

    @pl.when(j >= NI + NJ)
    def _():
        i = j - NI - NJ
        part = jax.lax.dot_general(a8t[pl.ds(i * BJ, BJ), :], g2b[...],
                                   (((1,), (0,)), ((), ())),
                                   preferred_element_type=jnp.float32)
        db = dis[0, pl.ds(i * BJ, BJ)]
        gj = g2b[pl.ds(i * BJ, BJ), :].astype(jnp.float32)
        h2c = jnp.maximum((part + gj) * db[:, None] + b2_ref[...], 0.0)
        of = jax.lax.dot_general(h2c, wfc_ref[...], (((1,), (1,)), ((), ())),
                                 preferred_element_type=jnp.float32)
        out_ref[0] = jnp.where(db[:, None] > 0, of + bfc_ref[...], 0.0)


def kernel(big_batch_positions, big_batched_adjacency_pruned, ego_mask_batch,
           W1, b1, W2, b2, Wfc, bfc):
    x = big_batch_positions
    A = big_batched_adjacency_pruned
    m = jnp.transpose(ego_mask_batch, (1, 0, 2)).reshape(T, 1, BN)
    m = m.astype(jnp.float32)
    b1r = b1.reshape(1, HID)
    b2r = b2.reshape(1, HID)
    bfcr = bfc.reshape(1, OUT)

    out = pl.pallas_call(
        _fused_kernel, grid=(T, NI + 2 * NJ),
        in_specs=[
            pl.BlockSpec((1, 1, BN), lambda t, j: (t, 0, 0)),
            pl.BlockSpec((1, BI, BN),
                         lambda t, j: (t, jnp.minimum(j, NI - 1), 0)),
            pl.BlockSpec((1, BN, IN_DIM), lambda t, j: (t, 0, 0)),
            pl.BlockSpec((HID, IN_DIM), lambda t, j: (0, 0)),
            pl.BlockSpec((1, HID), lambda t, j: (0, 0)),
            pl.BlockSpec((HID, HID), lambda t, j: (0, 0)),
            pl.BlockSpec((1, HID), lambda t, j: (0, 0)),
            pl.BlockSpec((OUT, HID), lambda t, j: (0, 0)),
            pl.BlockSpec((1, OUT), lambda t, j: (0, 0)),
        ],
        out_specs=pl.BlockSpec(
            (1, BJ, OUT),
            lambda t, j: (t, jnp.clip(j - NI - NJ, 0, NJ - 1), 0)),
        out_shape=jax.ShapeDtypeStruct((T, BN, OUT), jnp.float32),
        scratch_shapes=[
            pltpu.VMEM((BN, BN), jnp.bfloat16),
            pltpu.VMEM((BN, HID), jnp.bfloat16),
            pltpu.VMEM((BN, HID), jnp.bfloat16),
            pltpu.VMEM((1, BN), jnp.float32),
            pltpu.VMEM((1, BN), jnp.float32),
        ],
        compiler_params=pltpu.CompilerParams(
            dimension_semantics=("arbitrary", "arbitrary"),
            vmem_limit_bytes=63 * 1024 * 1024),
    )(m, A, x, W1, b1r, W2, b2r, Wfc, bfcr)

    h_stack = out.reshape(T, B, N, OUT)
    return jnp.transpose(h_stack, (1, 2, 0, 3))


# speedup vs baseline: 1.3436x; 1.0190x over previous
import jax
import jax.numpy as jnp
from jax.experimental import pallas as pl
from jax.experimental.pallas import tpu as pltpu

T, B, N = 4, 8, 512
BN = B * N
IN_DIM, HID, OUT = 128, 128, 64

BI = 512
NI = BN // BI
BJ = 2048
NJ = BN // BJ


def _fused_kernel(m_ref, a_ref, x_ref, w1_ref, b1_ref, w2_ref, b2_ref,
                  wfc_ref, bfc_ref, out_ref, a8t, g1b, g2b, deg, dis):
    j = pl.program_id(1)

    @pl.when(j < NI)
    def _():
        i = j
        a = a_ref[0]
        a8t[:, pl.ds(i * BI, BI)] = a.astype(jnp.bfloat16).T
        mi = m_ref[0, 0, pl.ds(i * BI, BI)]
        part = jnp.dot(mi[None, :], a, preferred_element_type=jnp.float32)

        @pl.when(i == 0)
        def _():
            deg[...] = part

        @pl.when(i > 0)
        def _():
            deg[...] += part

        @pl.when(i == NI - 1)
        def _():
            m = m_ref[0, 0]
            d = deg[0] * m + m
            dis[...] = jnp.where(d > 0, jax.lax.rsqrt(d), 0.0)[None]
            h = jax.lax.dot_general(x_ref[0], w1_ref[...],
                                    (((1,), (1,)), ((), ())),
                                    preferred_element_type=jnp.float32)
            g1b[...] = (h * dis[0][:, None]).astype(jnp.bfloat16)

    @pl.when(jnp.logical_and(j >= NI, j < NI + NJ))
    def _():
        i = j - NI
        part = jax.lax.dot_general(a8t[pl.ds(i * BJ, BJ), :], g1b[...],
                                   (((1,), (0,)), ((), ())),
                                   preferred_element_type=jnp.float32)
        db = dis[0, pl.ds(i * BJ, BJ)]
        gj = g1b[pl.ds(i * BJ, BJ), :].astype(jnp.float32)
        h1c = jnp.maximum((part + gj) * db[:, None] + b1_ref[...], 0.0)
        h2 = jax.lax.dot_general(h1c, w2_ref[...], (((1,), (1,)), ((), ())),
                                 preferred_element_type=jnp.float32)
        g2b[pl.ds(i * BJ, BJ), :] = (h2 * db[:, None]).astype(jnp.bfloat16)

    @pl.when(j >= NI + NJ)
    def _():
        i = j - NI - NJ
        part = jax.lax.dot_general(a8t[pl.ds(i * BJ, BJ), :], g2b[...],
                                   (((1,), (0,)), ((), ())),
                                   preferred_element_type=jnp.float32)
        db = dis[0, pl.ds(i * BJ, BJ)]
        gj = g2b[pl.ds(i * BJ, BJ), :].astype(jnp.float32)
        h2c = jnp.maximum((part + gj) * db[:, None] + b2_ref[...], 0.0)
        of = jax.lax.dot_general(h2c, wfc_ref[...], (((1,), (1,)), ((), ())),
                                 preferred_element_type=jnp.float32)
        out_ref[0] = jnp.where(db[:, None] > 0, of + bfc_ref[...], 0.0)


def kernel(big_batch_positions, big_batched_adjacency_pruned, ego_mask_batch,
           W1, b1, W2, b2, Wfc, bfc):
    x = big_batch_positions
    A = big_batched_adjacency_pruned
    m = jnp.transpose(ego_mask_batch, (1, 0, 2)).reshape(T, 1, BN)
    m = m.astype(jnp.float32)
    b1r = b1.reshape(1, HID)
    b2r = b2.reshape(1, HID)
    bfcr = bfc.reshape(1, OUT)

    out = pl.pallas_call(
        _fused_kernel, grid=(T, NI + 2 * NJ),
        in_specs=[
            pl.BlockSpec((1, 1, BN), lambda t, j: (t, 0, 0)),
            pl.BlockSpec((1, BI, BN),
                         lambda t, j: (t, jnp.minimum(j, NI - 1), 0)),
            pl.BlockSpec((1, BN, IN_DIM), lambda t, j: (t, 0, 0)),
            pl.BlockSpec((HID, IN_DIM), lambda t, j: (0, 0)),
            pl.BlockSpec((1, HID), lambda t, j: (0, 0)),
            pl.BlockSpec((HID, HID), lambda t, j: (0, 0)),
            pl.BlockSpec((1, HID), lambda t, j: (0, 0)),
            pl.BlockSpec((OUT, HID), lambda t, j: (0, 0)),
            pl.BlockSpec((1, OUT), lambda t, j: (0, 0)),
        ],
        out_specs=pl.BlockSpec(
            (1, BJ, OUT),
            lambda t, j: (t, jnp.clip(j - NI - NJ, 0, NJ - 1), 0)),
        out_shape=jax.ShapeDtypeStruct((T, BN, OUT), jnp.float32),
        scratch_shapes=[
            pltpu.VMEM((BN, BN), jnp.bfloat16),
            pltpu.VMEM((BN, HID), jnp.bfloat16),
            pltpu.VMEM((BN, HID), jnp.bfloat16),
            pltpu.VMEM((1, BN), jnp.float32),
            pltpu.VMEM((1, BN), jnp.float32),
        ],
        compiler_params=pltpu.CompilerParams(
            dimension_semantics=("arbitrary", "arbitrary"),
            vmem_limit_bytes=63 * 1024 * 1024),
    )(m, A, x, W1, b1r, W2, b2r, Wfc, bfcr)

    h_stack = out.reshape(T, B, N, OUT)
    return jnp.transpose(h_stack, (1, 2, 0, 3))


# speedup vs baseline: 1.4910x; 1.1097x over previous
import jax
import jax.numpy as jnp
from jax.experimental import pallas as pl
from jax.experimental.pallas import tpu as pltpu

T, B, N = 4, 8, 512
BN = B * N
IN_DIM, HID, OUT = 128, 128, 64

BI = 512
NI = BN // BI
BJ = 1024
NJ = BN // BJ
NP = NI + 1


def _quantize(i, m_ref, a_ref, a8t, deg):
    a = a_ref[0]
    a8t[:, pl.ds(i * BI, BI)] = a.astype(jnp.bfloat16).T.astype(jnp.int8)
    mi = m_ref[0, 0, pl.ds(i * BI, BI)]
    part = jnp.dot(mi[None, :], a, preferred_element_type=jnp.float32)

    @pl.when(i == 0)
    def _():
        deg[...] = part

    @pl.when(i > 0)
    def _():
        deg[...] += part


def _conv1_block(i, a8t, g1b, g2b, dis, b1_ref, w2_ref):
    part = jax.lax.dot_general(a8t[pl.ds(i * BJ, BJ), :], g1b[...],
                               (((1,), (0,)), ((), ())),
                               preferred_element_type=jnp.float32)
    db = dis[0, pl.ds(i * BJ, BJ)]
    gj = g1b[pl.ds(i * BJ, BJ), :].astype(jnp.float32)
    h1c = jnp.maximum((part + gj) * db[:, None] + b1_ref[...], 0.0)
    h2 = jax.lax.dot_general(h1c, w2_ref[...], (((1,), (1,)), ((), ())),
                             preferred_element_type=jnp.float32)
    g2b[pl.ds(i * BJ, BJ), :] = (h2 * db[:, None]).astype(jnp.bfloat16)


def _conv2_block(i, a8t, g2b, dis, b2_ref, wfc_ref, bfc_ref, out_ref):
    part = jax.lax.dot_general(a8t[pl.ds(i * BJ, BJ), :], g2b[...],
                               (((1,), (0,)), ((), ())),
                               preferred_element_type=jnp.float32)
    db = dis[0, pl.ds(i * BJ, BJ)]
    gj = g2b[pl.ds(i * BJ, BJ), :].astype(jnp.float32)
    h2c = jnp.maximum((part + gj) * db[:, None] + b2_ref[...], 0.0)
    of = jax.lax.dot_general(h2c, wfc_ref[...], (((1,), (1,)), ((), ())),
                             preferred_element_type=jnp.float32)
    out_ref[0] = jnp.where(db[:, None] > 0, of + bfc_ref[...], 0.0)


def _fused_kernel(m_ref, mp_ref, a_ref, x_ref, w1_ref, b1_ref, w2_ref,
                  b2_ref, wfc_ref, bfc_ref, out_ref,
                  a8tA, a8tB, g1b, g2b, deg, dis):
    u = pl.program_id(0)
    p = pl.program_id(1)

    @pl.when(jnp.logical_and(u > 0, p == 0))
    def _():
        m = mp_ref[0, 0]
        d = deg[0] * m + m
        dis[...] = jnp.where(d > 0, jax.lax.rsqrt(d), 0.0)[None]
        h = jax.lax.dot_general(x_ref[0], w1_ref[...],
                                (((1,), (1,)), ((), ())),
                                preferred_element_type=jnp.float32)
        g1b[...] = (h * dis[0][:, None]).astype(jnp.bfloat16)

    @pl.when(jnp.logical_and(u < T, p < NI))
    def _():
        @pl.when(u % 2 == 0)
        def _():
            _quantize(p, m_ref, a_ref, a8tA, deg)

        @pl.when(u % 2 == 1)
        def _():
            _quantize(p, m_ref, a_ref, a8tB, deg)

    @pl.when(jnp.logical_and(u > 0, jnp.logical_and(p >= 1, p <= NJ)))
    def _():
        i = p - 1

        @pl.when(u % 2 == 1)
        def _():
            _conv1_block(i, a8tA, g1b, g2b, dis, b1_ref, w2_ref)

        @pl.when(u % 2 == 0)
        def _():
            _conv1_block(i, a8tB, g1b, g2b, dis, b1_ref, w2_ref)

    @pl.when(jnp.logical_and(u > 0, p > NJ))
    def _():
        i = p - NJ - 1

        @pl.when(u % 2 == 1)
        def _():
            _conv2_block(i, a8tA, g2b, dis, b2_ref, wfc_ref, bfc_ref, out_ref)

        @pl.when(u % 2 == 0)
        def _():
            _conv2_block(i, a8tB, g2b, dis, b2_ref, wfc_ref, bfc_ref, out_ref)


def kernel(big_batch_positions, big_batched_adjacency_pruned, ego_mask_batch,
           W1, b1, W2, b2, Wfc, bfc):
    x = big_batch_positions
    A = big_batched_adjacency_pruned
    m = jnp.transpose(ego_mask_batch, (1, 0, 2)).reshape(T, 1, BN)
    m = m.astype(jnp.float32)
    b1r = b1.reshape(1, HID)
    b2r = b2.reshape(1, HID)
    bfcr = bfc.reshape(1, OUT)

    tq = lambda u: jnp.minimum(u, T - 1)
    tc = lambda u: jnp.maximum(u - 1, 0)

    out = pl.pallas_call(
        _fused_kernel, grid=(T + 1, NP),
        in_specs=[
            pl.BlockSpec((1, 1, BN), lambda u, p: (tq(u), 0, 0)),
            pl.BlockSpec((1, 1, BN), lambda u, p: (tc(u), 0, 0)),
            pl.BlockSpec((1, BI, BN),
                         lambda u, p: (tq(u), jnp.minimum(p, NI - 1), 0)),
            pl.BlockSpec((1, BN, IN_DIM), lambda u, p: (tc(u), 0, 0)),
            pl.BlockSpec((HID, IN_DIM), lambda u, p: (0, 0)),
            pl.BlockSpec((1, HID), lambda u, p: (0, 0)),
            pl.BlockSpec((HID, HID), lambda u, p: (0, 0)),
            pl.BlockSpec((1, HID), lambda u, p: (0, 0)),
            pl.BlockSpec((OUT, HID), lambda u, p: (0, 0)),
            pl.BlockSpec((1, OUT), lambda u, p: (0, 0)),
        ],
        out_specs=pl.BlockSpec(
            (1, BJ, OUT),
            lambda u, p: (tc(u), jnp.clip(p - NJ - 1, 0, NJ - 1), 0)),
        out_shape=jax.ShapeDtypeStruct((T, BN, OUT), jnp.float32),
        scratch_shapes=[
            pltpu.VMEM((BN, BN), jnp.int8),
            pltpu.VMEM((BN, BN), jnp.int8),
            pltpu.VMEM((BN, HID), jnp.bfloat16),
            pltpu.VMEM((BN, HID), jnp.bfloat16),
            pltpu.VMEM((1, BN), jnp.float32),
            pltpu.VMEM((1, BN), jnp.float32),
        ],
        compiler_params=pltpu.CompilerParams(
            dimension_semantics=("arbitrary", "arbitrary"),
            vmem_limit_bytes=63 * 1024 * 1024),
    )(m, m, A, x, W1, b1r, W2, b2r, Wfc, bfcr)

    h_stack = out.reshape(T, B, N, OUT)
    return jnp.transpose(h_stack, (1, 2, 0, 3))


# speedup vs baseline: 1.5585x; 1.0453x over previous
import jax
import jax.numpy as jnp
from jax.experimental import pallas as pl
from jax.experimental.pallas import tpu as pltpu

T, B, N = 4, 8, 512
BN = B * N
IN_DIM, HID, OUT = 128, 128, 64

BI = 512
NI = BN // BI
BJ = 1024
NJ = BN // BJ
NP = NI + 1


def _quantize(i, m_ref, a_ref, a8t, deg):
    a = a_ref[0]
    a8t[:, pl.ds(i * BI, BI)] = a.astype(jnp.int8).T
    mi = m_ref[0, 0, pl.ds(i * BI, BI)]
    part = jnp.dot(mi[None, :], a, preferred_element_type=jnp.float32)

    @pl.when(i == 0)
    def _():
        deg[...] = part

    @pl.when(i > 0)
    def _():
        deg[...] += part


def _conv1_block(i, a8t, g1b, g2b, dis, b1_ref, w2_ref):
    part = jax.lax.dot_general(a8t[pl.ds(i * BJ, BJ), :], g1b[...],
                               (((1,), (0,)), ((), ())),
                               preferred_element_type=jnp.float32)
    db = dis[0, pl.ds(i * BJ, BJ)]
    gj = g1b[pl.ds(i * BJ, BJ), :].astype(jnp.float32)
    h1c = jnp.maximum((part + gj) * db[:, None] + b1_ref[...], 0.0)
    h2 = jax.lax.dot_general(h1c, w2_ref[...], (((1,), (1,)), ((), ())),
                             preferred_element_type=jnp.float32)
    g2b[pl.ds(i * BJ, BJ), :] = (h2 * db[:, None]).astype(jnp.bfloat16)


def _conv2_block(i, a8t, g2b, dis, b2_ref, wfc_ref, bfc_ref, out_ref):
    part = jax.lax.dot_general(a8t[pl.ds(i * BJ, BJ), :], g2b[...],
                               (((1,), (0,)), ((), ())),
                               preferred_element_type=jnp.float32)
    db = dis[0, pl.ds(i * BJ, BJ)]
    gj = g2b[pl.ds(i * BJ, BJ), :].astype(jnp.float32)
    h2c = jnp.maximum((part + gj) * db[:, None] + b2_ref[...], 0.0)
    of = jax.lax.dot_general(h2c, wfc_ref[...], (((1,), (1,)), ((), ())),
                             preferred_element_type=jnp.float32)
    out_ref[0] = jnp.where(db[:, None] > 0, of + bfc_ref[...], 0.0)


def _fused_kernel(m_ref, mp_ref, a_ref, x_ref, w1_ref, b1_ref, w2_ref,
                  b2_ref, wfc_ref, bfc_ref, out_ref,
                  a8tA, a8tB, g1b, g2b, deg, dis):
    u = pl.program_id(0)
    p = pl.program_id(1)

    @pl.when(jnp.logical_and(u > 0, p == 0))
    def _():
        m = mp_ref[0, 0]
        d = deg[0] * m + m
        dis[...] = jnp.where(d > 0, jax.lax.rsqrt(d), 0.0)[None]
        h = jax.lax.dot_general(x_ref[0], w1_ref[...],
                                (((1,), (1,)), ((), ())),
                                preferred_element_type=jnp.float32)
        g1b[...] = (h * dis[0][:, None]).astype(jnp.bfloat16)

    @pl.when(jnp.logical_and(u < T, p < NI))
    def _():
        @pl.when(u % 2 == 0)
        def _():
            _quantize(p, m_ref, a_ref, a8tA, deg)

        @pl.when(u % 2 == 1)
        def _():
            _quantize(p, m_ref, a_ref, a8tB, deg)

    @pl.when(jnp.logical_and(u > 0, jnp.logical_and(p >= 1, p <= NJ)))
    def _():
        i = p - 1

        @pl.when(u % 2 == 1)
        def _():
            _conv1_block(i, a8tA, g1b, g2b, dis, b1_ref, w2_ref)

        @pl.when(u % 2 == 0)
        def _():
            _conv1_block(i, a8tB, g1b, g2b, dis, b1_ref, w2_ref)

    @pl.when(jnp.logical_and(u > 0, p > NJ))
    def _():
        i = p - NJ - 1

        @pl.when(u % 2 == 1)
        def _():
            _conv2_block(i, a8tA, g2b, dis, b2_ref, wfc_ref, bfc_ref, out_ref)

        @pl.when(u % 2 == 0)
        def _():
            _conv2_block(i, a8tB, g2b, dis, b2_ref, wfc_ref, bfc_ref, out_ref)


def kernel(big_batch_positions, big_batched_adjacency_pruned, ego_mask_batch,
           W1, b1, W2, b2, Wfc, bfc):
    x = big_batch_positions
    A = big_batched_adjacency_pruned
    m = jnp.transpose(ego_mask_batch, (1, 0, 2)).reshape(T, 1, BN)
    m = m.astype(jnp.float32)
    b1r = b1.reshape(1, HID)
    b2r = b2.reshape(1, HID)
    bfcr = bfc.reshape(1, OUT)

    tq = lambda u: jnp.minimum(u, T - 1)
    tc = lambda u: jnp.maximum(u - 1, 0)

    out = pl.pallas_call(
        _fused_kernel, grid=(T + 1, NP),
        in_specs=[
            pl.BlockSpec((1, 1, BN), lambda u, p: (tq(u), 0, 0)),
            pl.BlockSpec((1, 1, BN), lambda u, p: (tc(u), 0, 0)),
            pl.BlockSpec((1, BI, BN),
                         lambda u, p: (tq(u), jnp.minimum(p, NI - 1), 0)),
            pl.BlockSpec((1, BN, IN_DIM), lambda u, p: (tc(u), 0, 0)),
            pl.BlockSpec((HID, IN_DIM), lambda u, p: (0, 0)),
            pl.BlockSpec((1, HID), lambda u, p: (0, 0)),
            pl.BlockSpec((HID, HID), lambda u, p: (0, 0)),
            pl.BlockSpec((1, HID), lambda u, p: (0, 0)),
            pl.BlockSpec((OUT, HID), lambda u, p: (0, 0)),
            pl.BlockSpec((1, OUT), lambda u, p: (0, 0)),
        ],
        out_specs=pl.BlockSpec(
            (1, BJ, OUT),
            lambda u, p: (tc(u), jnp.clip(p - NJ - 1, 0, NJ - 1), 0)),
        out_shape=jax.ShapeDtypeStruct((T, BN, OUT), jnp.float32),
        scratch_shapes=[
            pltpu.VMEM((BN, BN), jnp.int8),
            pltpu.VMEM((BN, BN), jnp.int8),
            pltpu.VMEM((BN, HID), jnp.bfloat16),
            pltpu.VMEM((BN, HID), jnp.bfloat16),
            pltpu.VMEM((1, BN), jnp.float32),
            pltpu.VMEM((1, BN), jnp.float32),
        ],
        compiler_params=pltpu.CompilerParams(
            dimension_semantics=("arbitrary", "arbitrary"),
            vmem_limit_bytes=63 * 1024 * 1024),
    )(m, m, A, x, W1, b1r, W2, b2r, Wfc, bfcr)

    h_stack = out.reshape(T, B, N, OUT)
    return jnp.transpose(h_stack, (1, 2, 0, 3))
